# baseline (speedup 1.0000x reference)
import functools
import math

import jax
import jax.numpy as jnp
from jax import lax
from jax.experimental import pallas as pl
from jax.experimental.pallas import tpu as pltpu
from jax.experimental.pallas import tpu_sc as plsc

D_MODEL = 128
SCALE = math.sqrt(float(D_MODEL))

_info = plsc.get_sparse_core_info()
_NC = _info.num_cores
_NS = _info.num_subcores
_NW = _NC * _NS
_L = _info.num_lanes

C0 = 128
NG = 3
NO = 2
UNROLL = 6


@functools.lru_cache(maxsize=None)
def _build(R, S, V, D):
    assert R % _NW == 0 and C0 < S <= 2 * C0 and C0 % 8 == 0 and S % 8 == 0
    r_per_w = R // _NW
    n_chunks = 2 * r_per_w
    n_main = (n_chunks // UNROLL) * UNROLL
    sizes = (C0, S - C0)
    mesh = plsc.VectorSubcoreMesh(core_axis_name="c", subcore_axis_name="s")

    @functools.partial(
        pl.kernel,
        mesh=mesh,
        out_type=jax.ShapeDtypeStruct((R, S, D), jnp.float32),
        scratch_types=[
            pltpu.VMEM((r_per_w, S), jnp.int32),
            pltpu.SemaphoreType.DMA,
        ] + [pltpu.VMEM((C0, D), jnp.float32)] * (NG + NO)
          + [pltpu.SemaphoreType.DMA] * (NG + NO),
    )
    def emb_kernel(idx_hbm, table_hbm, out_hbm, idx_v, isem, *bufs_and_sems):
        gbuf = bufs_and_sems[:NG]
        obuf = bufs_and_sems[NG:NG + NO]
        gsem = bufs_and_sems[NG + NO:2 * NG + NO]
        osem = bufs_and_sems[2 * NG + NO:]
        wid = lax.axis_index("s") * _NC + lax.axis_index("c")
        row0 = wid * r_per_w
        pltpu.async_copy(idx_hbm.at[pl.ds(row0, r_per_w)], idx_v, isem).wait()
        scale_vec = jnp.full((_L,), SCALE, dtype=jnp.float32)

        def gather(ci, g, par):
            sz = sizes[par]
            pltpu.async_copy(
                table_hbm.at[idx_v.at[ci // 2, pl.ds(par * C0, sz)]],
                gbuf[g].at[pl.ds(0, sz)], gsem[g])

        def wait_gather(g, sz):
            pltpu.make_async_copy(out_hbm.at[0, pl.ds(0, sz)],
                                  gbuf[g].at[pl.ds(0, sz)], gsem[g]).wait()

        def wait_out(o, sz):
            pltpu.make_async_copy(obuf[o].at[pl.ds(0, sz)],
                                  out_hbm.at[0, pl.ds(0, sz)], osem[o]).wait()

        def scale(g, o, sz):
            src, dst = gbuf[g], obuf[o]

            def row_body(r2, c2):
                for rr in range(2):
                    r = r2 * 2 + rr
                    for j in range(D // _L):
                        dst[r, pl.ds(j * _L, _L)] = (
                            src[r, pl.ds(j * _L, _L)] * scale_vec)
                return c2

            lax.fori_loop(0, sz // 2, row_body, 0)

        def step(ci, k, first, fire):
            g, o = k % NG, k % NO
            par = k % 2
            sz = sizes[par]
            wait_gather(g, sz)
            if fire:
                gather(ci + NG - 1, (k + NG - 1) % NG, par)
            if not first:
                wait_out(o, sz)
            scale(g, o, sz)
            pltpu.async_copy(obuf[o].at[pl.ds(0, sz)],
                             out_hbm.at[row0 + ci // 2, pl.ds(par * C0, sz)],
                             osem[o])

        for ci in range(NG - 1):
            gather(ci, ci % NG, ci % 2)

        def outer(i, carry):
            ci0 = i * UNROLL
            for k in range(UNROLL):
                step(ci0 + k, k, first=False, fire=True)
            return carry

        for ci in range(UNROLL):
            step(ci, ci, first=ci < NO, fire=True)
        lax.fori_loop(1, n_main // UNROLL, outer, 0)
        for ci in range(n_main, n_chunks):
            step(ci, ci % UNROLL, first=False,
                 fire=ci + NG - 1 < n_chunks)

        for o in range(NO):
            wait_out(o, sizes[(n_chunks - NO + o) % 2])

    return emb_kernel


def kernel(x, table):
    R, S = x.shape
    V, D = table.shape
    return _build(R, S, V, D)(x.astype(jnp.int32), table)
